# baseline (speedup 1.0000x reference)
import functools

import jax
import jax.numpy as jnp
from jax import lax
from jax.experimental import pallas as pl
from jax.experimental.pallas import tpu as pltpu
from jax.experimental.pallas import tpu_sc as plsc

A = 6_400_000
NW = 32
CHUNK = 4_000
PER_W = A // NW
N_CHUNKS = PER_W // CHUNK

_C0 = 0.99998007
_C1 = -0.33269442
_C2 = 0.19401986
_C3 = -0.11769517
_C4 = 0.05408272
_C5 = -0.01229974

_HALF_PI = 1.5707963267948966
_PI = 3.141592653589793


def _angle_16(a, b, c, d, e, f):
    x = -(a * d + b * e + c * f)
    cx = b * f - c * e
    cy = c * d - a * f
    cz = a * e - b * d
    s = cx * cx + cy * cy + cz * cz
    i = lax.bitcast_convert_type(s, jnp.int32)
    i = jnp.int32(0x5F3759DF) - lax.shift_right_logical(i, 1)
    r = lax.bitcast_convert_type(i, jnp.float32)
    r = r * (1.5 - 0.5 * s * r * r)
    r = r * (1.5 - 0.5 * s * r * r)
    y = jnp.maximum(s * r, 1e-9)
    ax = jnp.abs(x)
    mn = jnp.minimum(ax, y)
    mx = jnp.maximum(ax, y)
    t = mn / mx
    t2 = t * t
    p = _C5
    p = p * t2 + _C4
    p = p * t2 + _C3
    p = p * t2 + _C2
    p = p * t2 + _C1
    p = p * t2 + _C0
    p = p * t
    base = jnp.where(ax > y, p, _HALF_PI - p)
    return jnp.where(x >= 0, base, _PI - base)


def _unpack_xy(w):
    i = lax.bitcast_convert_type(w, jnp.int32)
    xx = lax.bitcast_convert_type(
        jnp.bitwise_and(i, jnp.int32(-65536)), jnp.float32)
    yy = lax.bitcast_convert_type(lax.shift_left(i, 16), jnp.float32)
    return xx, yy


@functools.partial(
    pl.kernel,
    mesh=plsc.VectorSubcoreMesh(core_axis_name="c", subcore_axis_name="s"),
    out_type=jax.ShapeDtypeStruct((A,), jnp.float32),
    scratch_types=[
        pltpu.VMEM((4 * CHUNK,), jnp.int32),
        pltpu.VMEM((4 * CHUNK,), jnp.int32),
        pltpu.VMEM((4 * CHUNK,), jnp.float32),
        pltpu.VMEM((4 * CHUNK,), jnp.float32),
        pltpu.VMEM((CHUNK,), jnp.float32),
        pltpu.SemaphoreType.DMA,
        pltpu.SemaphoreType.DMA,
    ],
)
def _angle_sc(packed_hbm, idx0_hbm, idx1_hbm, out_hbm,
              idx_v0, idx_v1, w_v0, w_v1, out_v, sem0, sem1):
    wid = lax.axis_index("s") * 2 + lax.axis_index("c")
    w_base = wid * PER_W

    idx_bufs = (idx_v0, idx_v1)
    w_bufs = (w_v0, w_v1)
    sems = (sem0, sem1)

    def fire(k, slot):
        base = w_base + k * CHUNK
        idx = idx_bufs[slot]
        w = w_bufs[slot]
        sem = sems[slot]
        pltpu.sync_copy(idx0_hbm.at[pl.ds(base, CHUNK)],
                        idx.at[pl.ds(0, CHUNK)])
        pltpu.sync_copy(idx1_hbm.at[pl.ds(base, CHUNK)],
                        idx.at[pl.ds(CHUNK, CHUNK)])

        def widen(i, carry):
            sl = pl.ds(i * 16, 16)
            v = idx[sl]
            va = v + v
            idx[sl] = va
            idx[pl.ds(2 * CHUNK + i * 16, 16)] = va + 1
            return carry

        lax.fori_loop(0, (2 * CHUNK) // 16, widen, 0, unroll=4)
        pltpu.async_copy(packed_hbm.at[idx], w, sem)

    def drain(slot):
        pltpu.make_async_copy(
            packed_hbm.at[idx_bufs[slot]], w_bufs[slot], sems[slot]).wait()

    def compute_and_store(k, slot):
        w = w_bufs[slot]

        def compute(i, carry2):
            sl0 = pl.ds(i * 16, 16)
            sl1 = pl.ds(CHUNK + i * 16, 16)
            sl2 = pl.ds(2 * CHUNK + i * 16, 16)
            sl3 = pl.ds(3 * CHUNK + i * 16, 16)
            a, b = _unpack_xy(w[sl0])
            d, e = _unpack_xy(w[sl1])
            out_v[pl.ds(i * 16, 16)] = _angle_16(a, b, w[sl2], d, e, w[sl3])
            return carry2

        lax.fori_loop(0, CHUNK // 16, compute, 0, unroll=2)
        base = w_base + k * CHUNK
        pltpu.sync_copy(out_v, out_hbm.at[pl.ds(base, CHUNK)])

    fire(0, 0)

    def chunk_body(k, carry):
        for step in range(2):
            kk = 2 * k + step
            slot = step
            fire(kk + 1, 1 - slot)
            drain(slot)
            compute_and_store(kk, slot)
        return carry

    lax.fori_loop(0, N_CHUNKS // 2 - 1, chunk_body, 0)

    k_last = N_CHUNKS - 2
    fire(k_last + 1, 1)
    drain(0)
    compute_and_store(k_last, 0)
    drain(1)
    compute_and_store(k_last + 1, 1)


def kernel(nbr_vec, angle_nbr_idx):
    xb = lax.bitcast_convert_type(
        nbr_vec[:, 0].astype(jnp.bfloat16), jnp.uint16).astype(jnp.uint32)
    yb = lax.bitcast_convert_type(
        nbr_vec[:, 1].astype(jnp.bfloat16), jnp.uint16).astype(jnp.uint32)
    w0 = lax.bitcast_convert_type((xb << 16) | yb, jnp.float32)
    packed = jnp.stack([w0, nbr_vec[:, 2]], axis=1).reshape(-1)
    idx0 = angle_nbr_idx[:, 0]
    idx1 = angle_nbr_idx[:, 1]
    return _angle_sc(packed, idx0, idx1)


# speedup vs baseline: 4.6985x; 4.6985x over previous
import functools

import jax
import jax.numpy as jnp
from jax import lax
from jax.experimental import pallas as pl
from jax.experimental.pallas import tpu as pltpu
from jax.experimental.pallas import tpu_sc as plsc

A = 6_400_000
NW = 32
CHUNK = 4_000
PER_W = A // NW
N_CHUNKS = PER_W // CHUNK

_C0 = 0.99998007
_C1 = -0.33269442
_C2 = 0.19401986
_C3 = -0.11769517
_C4 = 0.05408272
_C5 = -0.01229974

_HALF_PI = 1.5707963267948966
_PI = 3.141592653589793


def _angle_16(a, b, c, d, e, f):
    x = -(a * d + b * e + c * f)
    cx = b * f - c * e
    cy = c * d - a * f
    cz = a * e - b * d
    s = cx * cx + cy * cy + cz * cz
    i = lax.bitcast_convert_type(s, jnp.int32)
    i = jnp.int32(0x5F3759DF) - lax.shift_right_logical(i, 1)
    r = lax.bitcast_convert_type(i, jnp.float32)
    r = r * (1.5 - 0.5 * s * r * r)
    r = r * (1.5 - 0.5 * s * r * r)
    y = jnp.maximum(s * r, 1e-9)
    ax = jnp.abs(x)
    mn = jnp.minimum(ax, y)
    mx = jnp.maximum(ax, y)
    t = mn / mx
    t2 = t * t
    p = _C5
    p = p * t2 + _C4
    p = p * t2 + _C3
    p = p * t2 + _C2
    p = p * t2 + _C1
    p = p * t2 + _C0
    p = p * t
    base = jnp.where(ax > y, p, _HALF_PI - p)
    return jnp.where(x >= 0, base, _PI - base)


def _unpack_xy(w):
    i = lax.bitcast_convert_type(w, jnp.int32)
    xx = lax.bitcast_convert_type(
        jnp.bitwise_and(i, jnp.int32(-65536)), jnp.float32)
    yy = lax.bitcast_convert_type(lax.shift_left(i, 16), jnp.float32)
    return xx, yy


@functools.partial(
    pl.kernel,
    mesh=plsc.VectorSubcoreMesh(core_axis_name="c", subcore_axis_name="s"),
    out_type=jax.ShapeDtypeStruct((A,), jnp.float32),
    scratch_types=[
        pltpu.VMEM((2 * CHUNK,), jnp.int32),
        pltpu.VMEM((2 * CHUNK,), jnp.int32),
        pltpu.VMEM((2 * CHUNK,), jnp.float32),
        pltpu.VMEM((2 * CHUNK,), jnp.float32),
        pltpu.VMEM((2 * CHUNK,), jnp.float32),
        pltpu.VMEM((2 * CHUNK,), jnp.float32),
        pltpu.VMEM((CHUNK,), jnp.float32),
        pltpu.SemaphoreType.DMA,
        pltpu.SemaphoreType.DMA,
    ],
)
def _angle_sc(wxy_hbm, wz_hbm, idx0_hbm, idx1_hbm, out_hbm,
              idx_v0, idx_v1, xy0, xy1, z0, z1, out_v, sem0, sem1):
    wid = lax.axis_index("s") * 2 + lax.axis_index("c")
    w_base = wid * PER_W

    idx_bufs = (idx_v0, idx_v1)
    xy_bufs = (xy0, xy1)
    z_bufs = (z0, z1)
    sems = (sem0, sem1)

    def fire(k, slot):
        base = w_base + k * CHUNK
        idx = idx_bufs[slot]
        sem = sems[slot]
        pltpu.sync_copy(idx0_hbm.at[pl.ds(base, CHUNK)],
                        idx.at[pl.ds(0, CHUNK)])
        pltpu.sync_copy(idx1_hbm.at[pl.ds(base, CHUNK)],
                        idx.at[pl.ds(CHUNK, CHUNK)])
        pltpu.async_copy(wxy_hbm.at[idx], xy_bufs[slot], sem)
        pltpu.async_copy(wz_hbm.at[idx], z_bufs[slot], sem)

    def drain(slot):
        idx = idx_bufs[slot]
        sem = sems[slot]
        pltpu.make_async_copy(wxy_hbm.at[idx], xy_bufs[slot], sem).wait()
        pltpu.make_async_copy(wz_hbm.at[idx], z_bufs[slot], sem).wait()

    def compute_and_store(k, slot):
        xy = xy_bufs[slot]
        z = z_bufs[slot]

        def compute(i, carry2):
            sl0 = pl.ds(i * 16, 16)
            sl1 = pl.ds(CHUNK + i * 16, 16)
            a, b = _unpack_xy(xy[sl0])
            d, e = _unpack_xy(xy[sl1])
            out_v[pl.ds(i * 16, 16)] = _angle_16(a, b, z[sl0], d, e, z[sl1])
            return carry2

        lax.fori_loop(0, CHUNK // 16, compute, 0, unroll=2)
        base = w_base + k * CHUNK
        pltpu.sync_copy(out_v, out_hbm.at[pl.ds(base, CHUNK)])

    fire(0, 0)

    def chunk_body(k, carry):
        for step in range(2):
            kk = 2 * k + step
            slot = step
            fire(kk + 1, 1 - slot)
            drain(slot)
            compute_and_store(kk, slot)
        return carry

    lax.fori_loop(0, N_CHUNKS // 2 - 1, chunk_body, 0)

    k_last = N_CHUNKS - 2
    fire(k_last + 1, 1)
    drain(0)
    compute_and_store(k_last, 0)
    drain(1)
    compute_and_store(k_last + 1, 1)


def kernel(nbr_vec, angle_nbr_idx):
    xb = lax.bitcast_convert_type(
        nbr_vec[:, 0].astype(jnp.bfloat16), jnp.uint16).astype(jnp.uint32)
    yb = lax.bitcast_convert_type(
        nbr_vec[:, 1].astype(jnp.bfloat16), jnp.uint16).astype(jnp.uint32)
    wxy = lax.bitcast_convert_type((xb << 16) | yb, jnp.float32)
    wz = nbr_vec[:, 2]
    idx0 = angle_nbr_idx[:, 0]
    idx1 = angle_nbr_idx[:, 1]
    return _angle_sc(wxy, wz, idx0, idx1)
